# baseline (speedup 1.0000x reference)
import functools

import jax
import jax.numpy as jnp
from jax import lax
from jax.experimental import pallas as pl
from jax.experimental.pallas import tpu as pltpu
from jax.experimental.pallas import tpu_sc as plsc

_B = 16384
_DIM = 128
_N = _B - 1
_SO = 8192
_NC = 2
_NS = 16
_NW = _NC * _NS
_RPW = _SO // _NW
_L = 16
_G = _DIM // _L
_CR = _RPW + 8

_TN = _B - _SO - 1
_TCH = 512
_NCH = (_B - _SO) // _TCH

_MESH = plsc.VectorSubcoreMesh(
    core_axis_name="c", subcore_axis_name="s", num_cores=_NC, num_subcores=_NS
)


@functools.partial(
    pl.kernel,
    out_type=[jax.ShapeDtypeStruct((_SO,), jnp.float32)] * 2,
    mesh=_MESH,
    compiler_params=pltpu.CompilerParams(needs_layout_passes=False),
    scratch_types=[
        pltpu.VMEM((_CR, _DIM), jnp.int32),
        pltpu.VMEM((_CR, _DIM), jnp.int32),
        pltpu.VMEM((8, _DIM), jnp.float32),
        pltpu.VMEM((_RPW,), jnp.float32),
        pltpu.VMEM((_RPW,), jnp.float32),
        pltpu.SemaphoreType.DMA,
        pltpu.SemaphoreType.DMA,
    ],
)
def _transe_sc(pos_hbm, neg_hbm, rel_hbm, pos_o, neg_o,
               buf_a, buf_b, r0_v, pos_v, neg_v, sem_a, sem_b):
    wid = lax.axis_index("c") * _NS + lax.axis_index("s")
    base = wid * _RPW

    pltpu.sync_copy(rel_hbm.at[pl.ds(0, 8)], r0_v)
    r0 = [r0_v[0, pl.ds(g * _L, _L)] for g in range(_G)]

    lanes = lax.iota(jnp.int32, _L)
    masks = [lanes == r for r in range(_L)]

    cp_a = pltpu.async_copy(pos_hbm.at[pl.ds(base, _CR)], buf_a, sem_a)
    cp_b = pltpu.async_copy(neg_hbm.at[pl.ds(base, _CR)], buf_b, sem_b)

    def compute(buf, out_v, swap):
        def load_row(k):
            return [buf[k, pl.ds(g * _L, _L)] for g in range(_G)]

        def grp(gi, prev):
            rb = gi * _L
            s_vec = jnp.zeros((_L,), jnp.float32)
            for r in range(_L):
                new = load_row(rb + r + 1)
                acc = None
                for g in range(_G):
                    dint = new[g] - prev[g] if swap else prev[g] - new[g]
                    d = dint.astype(jnp.float32) + r0[g]
                    acc = jnp.abs(d) if acc is None else acc + jnp.abs(d)
                s_vec = jnp.where(masks[r], jnp.sum(acc), s_vec)
                prev = new
            out_v[pl.ds(rb, _L)] = s_vec
            return tuple(prev)

        lax.fori_loop(0, _RPW // _L, grp, tuple(load_row(0)))

    cp_a.wait()
    compute(buf_a, pos_v, False)
    cp_b.wait()
    compute(buf_b, neg_v, True)

    pltpu.sync_copy(pos_v, pos_o.at[pl.ds(base, _RPW)])
    pltpu.sync_copy(neg_v, neg_o.at[pl.ds(base, _RPW)])


def _transe_tc_body(pos_hbm, neg_hbm, rel_hbm, pos_o, neg_o,
                    buf_a, buf_b, r0_v, rs_a, rs_b, sem_a, sem_b, sem_o):
    cp_r = pltpu.make_async_copy(rel_hbm.at[pl.ds(0, 8)], r0_v, sem_o)
    cp_r.start()
    cp_r.wait()
    r0row = r0_v[0:1, :]

    bufs = (buf_a, buf_b)
    sems = (sem_a, sem_b)
    ones_row = jnp.ones((1, _DIM), jnp.float32)

    def issue(idx):
        c = idx % _NCH
        start = _SO + c * _TCH
        rows = _TCH + 8 if c < _NCH - 1 else _TCH
        return pltpu.make_async_copy(
            (pos_hbm if idx < _NCH else neg_hbm).at[pl.ds(start, rows)],
            bufs[idx % 2].at[pl.ds(0, rows)], sems[idx % 2])

    rss = (rs_a, rs_b)

    def compute(idx):
        c = idx % _NCH
        buf = bufs[idx % 2]
        swap = idx >= _NCH
        cur = buf[pl.ds(0, _TCH), :]
        nxt = buf[pl.ds(1, _TCH), :]
        dint = nxt - cur if swap else cur - nxt
        a = jnp.abs(dint.astype(jnp.float32) + r0row)
        rs = lax.dot_general(ones_row, a, (((1,), (1,)), ((), ())),
                             preferred_element_type=jnp.float32)
        rss[idx % 2][...] = rs
        cp = pltpu.make_async_copy(
            rss[idx % 2].at[0],
            (pos_o if idx < _NCH else neg_o).at[pl.ds(c * _TCH, _TCH)], sem_o)
        cp.start()
        return cp

    n_total = 2 * _NCH
    cps = [issue(0), issue(1)]
    cps[0].start()
    cps[1].start()
    cps_out = []
    for idx in range(n_total):
        cps[idx].wait()
        if idx >= 2:
            cps_out[idx - 2].wait()
        cps_out.append(compute(idx))
        if idx + 2 < n_total:
            cp = issue(idx + 2)
            cp.start()
            cps.append(cp)
    cps_out[-2].wait()
    cps_out[-1].wait()


_transe_tc = pl.pallas_call(
    _transe_tc_body,
    out_shape=[jax.ShapeDtypeStruct((_B - _SO,), jnp.float32)] * 2,
    in_specs=[pl.BlockSpec(memory_space=pltpu.MemorySpace.HBM)] * 3,
    out_specs=[pl.BlockSpec(memory_space=pltpu.MemorySpace.HBM)] * 2,
    scratch_shapes=[
        pltpu.VMEM((_TCH + 8, _DIM), jnp.int32),
        pltpu.VMEM((_TCH + 8, _DIM), jnp.int32),
        pltpu.VMEM((8, _DIM), jnp.float32),
        pltpu.VMEM((1, _TCH), jnp.float32),
        pltpu.VMEM((1, _TCH), jnp.float32),
        pltpu.SemaphoreType.DMA,
        pltpu.SemaphoreType.DMA,
        pltpu.SemaphoreType.DMA,
    ],
)


def _assemble_body(pos_sc, neg_sc, pos_tc, neg_tc, loss_o, pos_o, neg_o,
                   pv4, nv4, ps, ns, ls, sem):
    cps = []
    for src, dst, off in (
        (pos_sc, pv4, 0), (neg_sc, nv4, 0),
        (pos_tc, pv4, _SO), (neg_tc, nv4, _SO),
    ):
        cp = pltpu.make_async_copy(src, dst.at[pl.ds(off, _SO)], sem)
        cp.start()
        cps.append(cp)
    for cp in cps:
        cp.wait()
    p = lax.slice(pv4[...], (0,), (_N,))
    n = lax.slice(nv4[...], (0,), (_N,))
    ps[...] = p
    ns[...] = n
    ls[...] = jnp.maximum(p - n + 1.0, 0.0)
    cps = []
    for src, dst in ((ls, loss_o), (ps, pos_o), (ns, neg_o)):
        cp = pltpu.make_async_copy(src, dst, sem)
        cp.start()
        cps.append(cp)
    for cp in cps:
        cp.wait()


_assemble = pl.pallas_call(
    _assemble_body,
    out_shape=[jax.ShapeDtypeStruct((_N,), jnp.float32)] * 3,
    in_specs=[pl.BlockSpec(memory_space=pltpu.MemorySpace.HBM)] * 4,
    out_specs=[pl.BlockSpec(memory_space=pltpu.MemorySpace.HBM)] * 3,
    scratch_shapes=[
        pltpu.VMEM((_B,), jnp.float32),
        pltpu.VMEM((_B,), jnp.float32),
        pltpu.VMEM((_N,), jnp.float32),
        pltpu.VMEM((_N,), jnp.float32),
        pltpu.VMEM((_N,), jnp.float32),
        pltpu.SemaphoreType.DMA,
    ],
)


def kernel(positive_triplets, negative_triplets, rel_weight):
    sc_pos, sc_neg = _transe_sc(positive_triplets, negative_triplets,
                                rel_weight)
    tc_pos, tc_neg = _transe_tc(positive_triplets, negative_triplets,
                                rel_weight)
    return _assemble(sc_pos, sc_neg, tc_pos, tc_neg)


# speedup vs baseline: 1.0724x; 1.0724x over previous
import functools

import jax
import jax.numpy as jnp
from jax import lax
from jax.experimental import pallas as pl
from jax.experimental.pallas import tpu as pltpu
from jax.experimental.pallas import tpu_sc as plsc

_B = 16384
_DIM = 128
_N = _B - 1
_SO = 6144
_NC = 2
_NS = 16
_NW = _NC * _NS
_RPW = _SO // _NW
_L = 16
_G = _DIM // _L
_CR = _RPW + 8

_TN = _B - _SO - 1
_TCH = 512
_NCH = (_B - _SO) // _TCH

_MESH = plsc.VectorSubcoreMesh(
    core_axis_name="c", subcore_axis_name="s", num_cores=_NC, num_subcores=_NS
)


@functools.partial(
    pl.kernel,
    out_type=[jax.ShapeDtypeStruct((_SO,), jnp.float32)] * 2,
    mesh=_MESH,
    compiler_params=pltpu.CompilerParams(needs_layout_passes=False),
    scratch_types=[
        pltpu.VMEM((_CR, _DIM), jnp.int32),
        pltpu.VMEM((_CR, _DIM), jnp.int32),
        pltpu.VMEM((8, _DIM), jnp.float32),
        pltpu.VMEM((_RPW,), jnp.float32),
        pltpu.VMEM((_RPW,), jnp.float32),
        pltpu.SemaphoreType.DMA,
        pltpu.SemaphoreType.DMA,
    ],
)
def _transe_sc(pos_hbm, neg_hbm, rel_hbm, pos_o, neg_o,
               buf_a, buf_b, r0_v, pos_v, neg_v, sem_a, sem_b):
    wid = lax.axis_index("c") * _NS + lax.axis_index("s")
    base = wid * _RPW

    pltpu.sync_copy(rel_hbm.at[pl.ds(0, 8)], r0_v)
    r0 = [r0_v[0, pl.ds(g * _L, _L)] for g in range(_G)]

    lanes = lax.iota(jnp.int32, _L)
    masks = [lanes == r for r in range(_L)]

    cp_a = pltpu.async_copy(pos_hbm.at[pl.ds(base, _CR)], buf_a, sem_a)
    cp_b = pltpu.async_copy(neg_hbm.at[pl.ds(base, _CR)], buf_b, sem_b)

    def compute(buf, out_v, swap):
        def load_row(k):
            return [buf[k, pl.ds(g * _L, _L)] for g in range(_G)]

        def grp(gi, prev):
            rb = gi * _L
            s_vec = jnp.zeros((_L,), jnp.float32)
            for r in range(_L):
                new = load_row(rb + r + 1)
                acc = None
                for g in range(_G):
                    dint = new[g] - prev[g] if swap else prev[g] - new[g]
                    d = dint.astype(jnp.float32) + r0[g]
                    acc = jnp.abs(d) if acc is None else acc + jnp.abs(d)
                s_vec = jnp.where(masks[r], jnp.sum(acc), s_vec)
                prev = new
            out_v[pl.ds(rb, _L)] = s_vec
            return tuple(prev)

        lax.fori_loop(0, _RPW // _L, grp, tuple(load_row(0)))

    cp_a.wait()
    compute(buf_a, pos_v, False)
    cp_b.wait()
    compute(buf_b, neg_v, True)

    pltpu.sync_copy(pos_v, pos_o.at[pl.ds(base, _RPW)])
    pltpu.sync_copy(neg_v, neg_o.at[pl.ds(base, _RPW)])


_NBUF = 4


def _transe_tc_body(pos_hbm, neg_hbm, rel_hbm, pos_o, neg_o,
                    buf_a, buf_b, buf_c, buf_d, r0_v, rs_a, rs_b,
                    sem_a, sem_b, sem_c, sem_d, sem_o):
    cp_r = pltpu.make_async_copy(rel_hbm.at[pl.ds(0, 8)], r0_v, sem_o)
    cp_r.start()
    cp_r.wait()
    r0row = r0_v[0:1, :]

    bufs = (buf_a, buf_b, buf_c, buf_d)
    sems = (sem_a, sem_b, sem_c, sem_d)
    ones_row = jnp.ones((1, _DIM), jnp.float32)

    def issue(idx):
        c = idx % _NCH
        start = _SO + c * _TCH
        rows = _TCH + 8 if c < _NCH - 1 else _TCH
        return pltpu.make_async_copy(
            (pos_hbm if idx < _NCH else neg_hbm).at[pl.ds(start, rows)],
            bufs[idx % _NBUF].at[pl.ds(0, rows)], sems[idx % _NBUF])

    rss = (rs_a, rs_b)

    def compute(idx):
        c = idx % _NCH
        buf = bufs[idx % _NBUF]
        swap = idx >= _NCH
        cur = buf[pl.ds(0, _TCH), :]
        nxt = buf[pl.ds(1, _TCH), :]
        dint = nxt - cur if swap else cur - nxt
        a = jnp.abs(dint.astype(jnp.float32) + r0row)
        rs = lax.dot_general(ones_row, a, (((1,), (1,)), ((), ())),
                             preferred_element_type=jnp.float32)
        rss[idx % 2][...] = rs
        cp = pltpu.make_async_copy(
            rss[idx % 2].at[0],
            (pos_o if idx < _NCH else neg_o).at[pl.ds(c * _TCH, _TCH)], sem_o)
        cp.start()
        return cp

    n_total = 2 * _NCH
    cps = [issue(i) for i in range(_NBUF)]
    for cp in cps:
        cp.start()
    cps_out = []
    for idx in range(n_total):
        cps[idx].wait()
        if idx >= 2:
            cps_out[idx - 2].wait()
        cps_out.append(compute(idx))
        if idx + _NBUF < n_total:
            cp = issue(idx + _NBUF)
            cp.start()
            cps.append(cp)
    cps_out[-2].wait()
    cps_out[-1].wait()


_transe_tc = pl.pallas_call(
    _transe_tc_body,
    out_shape=[jax.ShapeDtypeStruct((_B - _SO,), jnp.float32)] * 2,
    in_specs=[pl.BlockSpec(memory_space=pltpu.MemorySpace.HBM)] * 3,
    out_specs=[pl.BlockSpec(memory_space=pltpu.MemorySpace.HBM)] * 2,
    scratch_shapes=[
        pltpu.VMEM((_TCH + 8, _DIM), jnp.int32),
        pltpu.VMEM((_TCH + 8, _DIM), jnp.int32),
        pltpu.VMEM((_TCH + 8, _DIM), jnp.int32),
        pltpu.VMEM((_TCH + 8, _DIM), jnp.int32),
        pltpu.VMEM((8, _DIM), jnp.float32),
        pltpu.VMEM((1, _TCH), jnp.float32),
        pltpu.VMEM((1, _TCH), jnp.float32),
        pltpu.SemaphoreType.DMA,
        pltpu.SemaphoreType.DMA,
        pltpu.SemaphoreType.DMA,
        pltpu.SemaphoreType.DMA,
        pltpu.SemaphoreType.DMA,
    ],
)


def _assemble_body(pos_sc, neg_sc, pos_tc, neg_tc, loss_o, pos_o, neg_o,
                   pv4, nv4, ps, ns, ls, sem):
    cps = []
    for src, dst, off, ln in (
        (pos_sc, pv4, 0, _SO), (neg_sc, nv4, 0, _SO),
        (pos_tc, pv4, _SO, _B - _SO), (neg_tc, nv4, _SO, _B - _SO),
    ):
        cp = pltpu.make_async_copy(src, dst.at[pl.ds(off, ln)], sem)
        cp.start()
        cps.append(cp)
    for cp in cps:
        cp.wait()
    p = lax.slice(pv4[...], (0,), (_N,))
    n = lax.slice(nv4[...], (0,), (_N,))
    ps[...] = p
    ns[...] = n
    ls[...] = jnp.maximum(p - n + 1.0, 0.0)
    cps = []
    for src, dst in ((ls, loss_o), (ps, pos_o), (ns, neg_o)):
        cp = pltpu.make_async_copy(src, dst, sem)
        cp.start()
        cps.append(cp)
    for cp in cps:
        cp.wait()


_assemble = pl.pallas_call(
    _assemble_body,
    out_shape=[jax.ShapeDtypeStruct((_N,), jnp.float32)] * 3,
    in_specs=[pl.BlockSpec(memory_space=pltpu.MemorySpace.HBM)] * 4,
    out_specs=[pl.BlockSpec(memory_space=pltpu.MemorySpace.HBM)] * 3,
    scratch_shapes=[
        pltpu.VMEM((_B,), jnp.float32),
        pltpu.VMEM((_B,), jnp.float32),
        pltpu.VMEM((_N,), jnp.float32),
        pltpu.VMEM((_N,), jnp.float32),
        pltpu.VMEM((_N,), jnp.float32),
        pltpu.SemaphoreType.DMA,
    ],
)


def kernel(positive_triplets, negative_triplets, rel_weight):
    sc_pos, sc_neg = _transe_sc(positive_triplets, negative_triplets,
                                rel_weight)
    tc_pos, tc_neg = _transe_tc(positive_triplets, negative_triplets,
                                rel_weight)
    return _assemble(sc_pos, sc_neg, tc_pos, tc_neg)
